# Initial kernel scaffold; baseline (speedup 1.0000x reference)
import jax
import jax.numpy as jnp
from jax.experimental import pallas as pl


def kernel(x, edge_index, W1, b1, W2, b2):
    raise NotImplementedError("write your pallas kernel here")


# speedup vs baseline: 7.6604x; 7.6604x over previous
import jax
import jax.numpy as jnp
from jax import lax
from jax.experimental import pallas as pl
from jax.experimental.pallas import tpu as pltpu
from jax.experimental.pallas import tpu_sc as plsc

N = 10000
E = 160000
D = 256
HALF = 128
CLS = 16
NC = 2
NS = 16
L = 16

NPAD = 10240
RPT = NPAD // NS
EPAD = 163840
CH1 = EPAD // NS // 128
CH2 = EPAD // (NC * NS) // 128

BN = 1024
NB = NPAD // BN


def _mesh():
    return plsc.VectorSubcoreMesh(core_axis_name="c", subcore_axis_name="s")


def _deg_body(dst_hbm, degp_hbm, idxv, dloc2):
    c = lax.axis_index("c")
    s = lax.axis_index("s")
    pltpu.sync_copy(dst_hbm.at[c, s], idxv)
    zeros16 = jnp.zeros((L,), jnp.float32)

    def zero_body(i, carry):
        dloc2[i // 8, pl.ds((i % 8) * L, L)] = zeros16
        return carry

    lax.fori_loop(0, NPAD // L, zero_body, 0)
    ones16 = jnp.ones((L,), jnp.float32)

    def hist_body(i, carry):
        j = i // 8
        k = i % 8
        idx = idxv[j, pl.ds(k * L, L)]
        plsc.addupdate_scatter(dloc2, [idx >> 7, idx & 127], ones16)
        return carry

    lax.fori_loop(0, CH2 * 8, hist_body, 0)
    pltpu.sync_copy(dloc2, degp_hbm.at[c, s])


def _deg_call(dst2):
    return pl.kernel(
        _deg_body,
        out_type=jax.ShapeDtypeStruct((NC, NS, NPAD // 128, 128), jnp.float32),
        mesh=_mesh(),
        compiler_params=pltpu.CompilerParams(needs_layout_passes=False),
        scratch_types=[
            pltpu.VMEM((CH2, 128), jnp.int32),
            pltpu.VMEM((NPAD // 128, 128), jnp.float32),
        ],
    )(dst2)


def _g1_body(x_ref, w1_ref, degp_ref, g_ref, dinv_ref):
    deg = jnp.sum(degp_ref[...], axis=0) + 1.0
    dinv = lax.rsqrt(deg)
    h = jnp.dot(x_ref[...], w1_ref[...], preferred_element_type=jnp.float32)
    g = h * dinv
    g_ref[0] = g[:, :HALF]
    g_ref[1] = g[:, HALF:]
    dinv_ref[...] = dinv


def _g1_call(x_pad, W1, degp3):
    return pl.pallas_call(
        _g1_body,
        grid=(NB,),
        in_specs=[
            pl.BlockSpec((BN, D), lambda i: (i, 0)),
            pl.BlockSpec((D, D), lambda i: (0, 0)),
            pl.BlockSpec((NC * NS, BN, 1), lambda i: (0, i, 0)),
        ],
        out_specs=[
            pl.BlockSpec((NC, BN, HALF), lambda i: (0, i, 0)),
            pl.BlockSpec((BN, 1), lambda i: (i, 0)),
        ],
        out_shape=[
            jax.ShapeDtypeStruct((NC, NPAD, HALF), jnp.float32),
            jax.ShapeDtypeStruct((NPAD, 1), jnp.float32),
        ],
    )(x_pad, W1, degp3)


GROUP = 16
NGRP = CH1 // GROUP


def _scat1_body(g_hbm, src_hbm, dst_hbm, out_hbm,
                srcv, dstv, rows0, rows1, acc, sem0, sem1):
    c = lax.axis_index("c")
    s = lax.axis_index("s")
    zeros16 = jnp.zeros((L,), jnp.float32)

    def zb(i, carry):
        rows0[i // (HALF // L), pl.ds((i % (HALF // L)) * L, L)] = zeros16
        return carry

    lax.fori_loop(0, 128 * (HALF // L), zb, 0)
    for q in range(RPT // 128):
        pltpu.sync_copy(rows0, acc.at[pl.ds(s * RPT + q * 128, 128)])
    plsc.subcore_barrier()

    def group(gi, carry):
        pltpu.sync_copy(src_hbm.at[c, s, pl.ds(gi * GROUP, GROUP)], srcv)
        pltpu.sync_copy(dst_hbm.at[s, pl.ds(gi * GROUP, GROUP)], dstv)
        pltpu.async_copy(g_hbm.at[srcv.at[0]], rows0, sem0)

        def step(j2, carry2):
            j = j2 * 2
            pltpu.make_async_copy(g_hbm.at[srcv.at[j]], rows0, sem0).wait()
            pltpu.async_copy(g_hbm.at[srcv.at[j + 1]], rows1, sem1)
            pltpu.sync_copy(rows0, acc.at[dstv.at[j]], add=True)
            pltpu.make_async_copy(g_hbm.at[srcv.at[j + 1]], rows1, sem1).wait()

            @pl.when(j + 2 < GROUP)
            def _():
                pltpu.async_copy(g_hbm.at[srcv.at[j + 2]], rows0, sem0)

            pltpu.sync_copy(rows1, acc.at[dstv.at[j + 1]], add=True)
            return carry2

        lax.fori_loop(0, GROUP // 2, step, 0)
        return carry

    lax.fori_loop(0, NGRP, group, 0)
    plsc.subcore_barrier()
    pltpu.sync_copy(acc.at[pl.ds(s * RPT, RPT)],
                    out_hbm.at[c, pl.ds(s * RPT, RPT)])


def _scat1_call(g_flat, src1o, dst1):
    return pl.kernel(
        _scat1_body,
        out_type=jax.ShapeDtypeStruct((NC, NPAD, HALF), jnp.float32),
        mesh=_mesh(),
        scratch_types=[
            pltpu.VMEM((GROUP, 128), jnp.int32),
            pltpu.VMEM((GROUP, 128), jnp.int32),
            pltpu.VMEM((128, HALF), jnp.float32),
            pltpu.VMEM((128, HALF), jnp.float32),
            pltpu.VMEM_SHARED((NPAD, HALF), jnp.float32),
            pltpu.SemaphoreType.DMA,
            pltpu.SemaphoreType.DMA,
        ],
    )(g_flat, src1o, dst1)


def _h2_body(scat_ref, g_ref, dinv_ref, b1_ref, w2_ref, g2_ref):
    m = scat_ref[...] + g_ref[...]
    h = jnp.concatenate([m[0], m[1]], axis=1)
    dinv = dinv_ref[...]
    o1 = jnp.maximum(h * dinv + b1_ref[...], 0.0)
    g2_ref[...] = jnp.dot(o1, w2_ref[...], preferred_element_type=jnp.float32) * dinv


def _h2_call(scat, g, dinv, b1r, W2):
    return pl.pallas_call(
        _h2_body,
        grid=(NB,),
        in_specs=[
            pl.BlockSpec((NC, BN, HALF), lambda i: (0, i, 0)),
            pl.BlockSpec((NC, BN, HALF), lambda i: (0, i, 0)),
            pl.BlockSpec((BN, 1), lambda i: (i, 0)),
            pl.BlockSpec((1, D), lambda i: (0, 0)),
            pl.BlockSpec((D, CLS), lambda i: (0, 0)),
        ],
        out_specs=pl.BlockSpec((BN, CLS), lambda i: (i, 0)),
        out_shape=jax.ShapeDtypeStruct((NPAD, CLS), jnp.float32),
    )(scat, g, dinv, b1r, W2)


def _scat2_body(g2_hbm, src_hbm, dst_hbm, out_hbm,
                srcv, dstv, rows0, rows1, acc, sem0, sem1):
    c = lax.axis_index("c")
    s = lax.axis_index("s")
    zeros16 = jnp.zeros((L,), jnp.float32)

    def zb(i, carry):
        rows0[i, pl.ds(0, L)] = zeros16
        return carry

    lax.fori_loop(0, 128, zb, 0)
    for q in range(RPT // 128):
        pltpu.sync_copy(rows0, acc.at[pl.ds(s * RPT + q * 128, 128)])
    pltpu.sync_copy(src_hbm.at[c, s], srcv)
    pltpu.sync_copy(dst_hbm.at[c, s], dstv)
    plsc.subcore_barrier()

    pltpu.async_copy(g2_hbm.at[srcv.at[0]], rows0, sem0)

    def step(j2, carry):
        j = j2 * 2
        pltpu.make_async_copy(g2_hbm.at[srcv.at[j]], rows0, sem0).wait()
        pltpu.async_copy(g2_hbm.at[srcv.at[j + 1]], rows1, sem1)
        pltpu.sync_copy(rows0, acc.at[dstv.at[j]], add=True)
        pltpu.make_async_copy(g2_hbm.at[srcv.at[j + 1]], rows1, sem1).wait()

        @pl.when(j + 2 < CH2)
        def _():
            pltpu.async_copy(g2_hbm.at[srcv.at[j + 2]], rows0, sem0)

        pltpu.sync_copy(rows1, acc.at[dstv.at[j + 1]], add=True)
        return carry

    lax.fori_loop(0, CH2 // 2, step, 0)
    plsc.subcore_barrier()
    pltpu.sync_copy(acc.at[pl.ds(s * RPT, RPT)],
                    out_hbm.at[c, pl.ds(s * RPT, RPT)])


def _scat2_call(g2, src2, dst2):
    return pl.kernel(
        _scat2_body,
        out_type=jax.ShapeDtypeStruct((NC, NPAD, CLS), jnp.float32),
        mesh=_mesh(),
        compiler_params=pltpu.CompilerParams(use_tc_tiling_on_sc=False),
        scratch_types=[
            pltpu.VMEM((CH2, 128), jnp.int32),
            pltpu.VMEM((CH2, 128), jnp.int32),
            pltpu.VMEM((128, CLS), jnp.float32),
            pltpu.VMEM((128, CLS), jnp.float32),
            pltpu.VMEM_SHARED((NPAD, CLS), jnp.float32),
            pltpu.SemaphoreType.DMA,
            pltpu.SemaphoreType.DMA,
        ],
    )(g2, src2, dst2)


def _fin_body(p2_ref, g2_ref, dinv_ref, b2_ref, out_ref):
    agg = jnp.sum(p2_ref[...], axis=0) + g2_ref[...]
    out_ref[...] = agg * dinv_ref[...] + b2_ref[...]


def _fin_call(p2, g2, dinv, b2r):
    return pl.pallas_call(
        _fin_body,
        grid=(NB,),
        in_specs=[
            pl.BlockSpec((NC, BN, CLS), lambda i: (0, i, 0)),
            pl.BlockSpec((BN, CLS), lambda i: (i, 0)),
            pl.BlockSpec((BN, 1), lambda i: (i, 0)),
            pl.BlockSpec((1, CLS), lambda i: (0, 0)),
        ],
        out_specs=pl.BlockSpec((BN, CLS), lambda i: (i, 0)),
        out_shape=jax.ShapeDtypeStruct((NPAD, CLS), jnp.float32),
    )(p2, g2, dinv, b2r)


def kernel(x, edge_index, W1, b1, W2, b2):
    src = edge_index[0].astype(jnp.int32)
    dst = edge_index[1].astype(jnp.int32)
    pad_e = EPAD - E
    src_p = jnp.concatenate([src, jnp.zeros((pad_e,), jnp.int32)])
    dst_p = jnp.concatenate([dst, jnp.full((pad_e,), N, jnp.int32)])
    src1 = src_p.reshape(NS, CH1, 128)
    src1o = jnp.stack([src1, src1 + NPAD])
    dst1 = dst_p.reshape(NS, CH1, 128)
    src2 = src_p.reshape(NC, NS, CH2, 128)
    dst2 = dst_p.reshape(NC, NS, CH2, 128)
    x_pad = jnp.pad(x, ((0, NPAD - N), (0, 0)))
    b1r = b1.reshape(1, D)
    b2r = b2.reshape(1, CLS)

    degp = _deg_call(dst2)
    degp3 = degp.reshape(NC * NS, NPAD, 1)
    g, dinv = _g1_call(x_pad, W1, degp3)
    g_flat = g.reshape(NC * NPAD, HALF)
    scat = _scat1_call(g_flat, src1o, dst1)
    g2 = _h2_call(scat, g, dinv, b1r, W2)
    p2 = _scat2_call(g2, src2, dst2)
    out = _fin_call(p2, g2, dinv, b2r)
    return out[:N]
